# baseline (speedup 1.0000x reference)
import functools

import jax
import jax.numpy as jnp
import numpy as np
from jax import lax
from jax.experimental import pallas as pl
from jax.experimental.pallas import tpu as pltpu
from jax.experimental.pallas import tpu_sc as plsc

_VOCAB = 100000
_D = 128
_SEQ = 2048
_BATCH = 4
_NC = 2
_NS = 16
_NW = _NC * _NS
_ROWS = (_BATCH * _SEQ) // _NW


def _pe_args() -> tuple[np.ndarray, np.ndarray]:
    div = np.exp(np.arange(0, _D, 2, dtype=np.float32) * (-np.log(10000.0) / _D))
    mul = np.repeat(div, 2)
    is_cos = (np.arange(_D) % 2).astype(np.float32)
    return mul, is_cos


_PE_MUL, _PE_IS_COS = _pe_args()


def _pe_on_device():
    pos = lax.iota(jnp.float32, _SEQ)[:, None]
    phase = pos * jnp.asarray(_PE_MUL)[None, :]
    return jnp.where(jnp.asarray(_PE_IS_COS)[None, :] > 0.5,
                     jnp.cos(phase), jnp.sin(phase))


_G = 4
_C = _ROWS // _G


def _sc_body(x_hbm, pe_hbm, table_hbm, out_hbm,
             idx_v, gb0, gb1, rb0, rb1, pe_v, sg0, sg1, so0, so1):
    wid = lax.axis_index("s") * _NC + lax.axis_index("c")
    base = wid * _ROWS
    batch = wid // (_SEQ // _ROWS)
    col = lax.rem(base, _SEQ)
    pltpu.sync_copy(x_hbm.at[batch, pl.ds(col, _ROWS)], idx_v)
    gbufs, rbufs = (gb0, gb1), (rb0, rb1)
    sgs, sos = (sg0, sg1), (so0, so1)
    gathers = [None] * _G
    scatters = [None] * _G
    gathers[0] = pltpu.async_copy(
        table_hbm.at[idx_v.at[pl.ds(0, _C)]], gb0, sg0)
    pe_base = lax.rem(base, _SEQ)
    pltpu.sync_copy(pe_hbm.at[pl.ds(pe_base, _ROWS)], pe_v)
    gathers[1] = pltpu.async_copy(
        table_hbm.at[idx_v.at[pl.ds(_C, _C)]], gb1, sg1)
    for g in range(_G):
        b = g % 2
        gathers[g].wait()
        if g >= 2:
            scatters[g - 2].wait()
        gb, rb = gbufs[b], rbufs[b]
        off = g * _C

        @plsc.parallel_loop(0, _C, unroll=4)
        def add_row(i, gb=gb, rb=rb, off=off):
            for c in range(_D // 16):
                sl = pl.ds(c * 16, 16)
                rb[i, sl] = gb[i, sl] + pe_v[off + i, sl]

        if g + 2 < _G:
            gathers[g + 2] = pltpu.async_copy(
                table_hbm.at[idx_v.at[pl.ds((g + 2) * _C, _C)]],
                gbufs[b], sgs[b])
        scatters[g] = pltpu.async_copy(
            rb, out_hbm.at[pl.ds(base + off, _C)], sos[b])
    scatters[_G - 2].wait()
    scatters[_G - 1].wait()


@functools.partial(jax.jit, static_argnames=())
def _run(x2d, table):
    pe = _pe_on_device()
    mesh = plsc.VectorSubcoreMesh(core_axis_name="c", subcore_axis_name="s")
    f = pl.kernel(
        _sc_body,
        mesh=mesh,
        out_type=jax.ShapeDtypeStruct((_BATCH * _SEQ, _D), jnp.float32),
        scratch_types=[
            pltpu.VMEM((_ROWS,), jnp.int32),
            pltpu.VMEM((_C, _D), jnp.float32),
            pltpu.VMEM((_C, _D), jnp.float32),
            pltpu.VMEM((_C, _D), jnp.float32),
            pltpu.VMEM((_C, _D), jnp.float32),
            pltpu.VMEM((_ROWS, _D), jnp.float32),
            pltpu.SemaphoreType.DMA,
            pltpu.SemaphoreType.DMA,
            pltpu.SemaphoreType.DMA,
            pltpu.SemaphoreType.DMA,
        ],
    )
    return f(x2d, pe, table)


def kernel(x, table):
    out = _run(x, table)
    return out.reshape(_BATCH, _SEQ, _D)


# speedup vs baseline: 1.1179x; 1.1179x over previous
import functools

import jax
import jax.numpy as jnp
import numpy as np
from jax import lax
from jax.experimental import pallas as pl
from jax.experimental.pallas import tpu as pltpu
from jax.experimental.pallas import tpu_sc as plsc

_VOCAB = 100000
_D = 128
_SEQ = 2048
_BATCH = 4
_NC = 2
_NS = 16
_NW = _NC * _NS
_ROWS = (_BATCH * _SEQ) // _NW


def _pe_table() -> np.ndarray:
    pos = np.arange(_SEQ, dtype=np.float32)[:, None]
    div = np.exp(np.arange(0, _D, 2, dtype=np.float32) * (-np.log(10000.0) / _D))
    pe = np.zeros((_SEQ, _D), dtype=np.float32)
    pe[:, 0::2] = np.sin(pos * div)
    pe[:, 1::2] = np.cos(pos * div)
    return pe


_PE = _pe_table()


_G = 4
_C = _ROWS // _G


def _sc_body(x_hbm, pe_hbm, table_hbm, out_hbm,
             idx_v, gb0, gb1, gb2, gb3, pe_v,
             sp, sg0, sg1, sg2, sg3, so0, so1, so2, so3):
    wid = lax.axis_index("s") * _NC + lax.axis_index("c")
    base = wid * _ROWS
    batch = wid // (_SEQ // _ROWS)
    col = lax.rem(base, _SEQ)
    gbufs = (gb0, gb1, gb2, gb3)
    sgs = (sg0, sg1, sg2, sg3)
    sos = (so0, so1, so2, so3)
    pe_load = pltpu.async_copy(pe_hbm.at[pl.ds(col, _ROWS)], pe_v, sp)
    pltpu.sync_copy(x_hbm.at[batch, pl.ds(col, _ROWS)], idx_v)
    gathers = [
        pltpu.async_copy(
            table_hbm.at[idx_v.at[pl.ds(g * _C, _C)]], gbufs[g], sgs[g])
        for g in range(_G)
    ]
    pe_load.wait()
    scatters = []
    for g in range(_G):
        gathers[g].wait()
        gb = gbufs[g]
        off = g * _C

        @plsc.parallel_loop(0, _C, unroll=4)
        def add_row(i, gb=gb, off=off):
            for c in range(_D // 16):
                sl = pl.ds(c * 16, 16)
                plsc.addupdate(gb.at[i, sl], pe_v[off + i, sl])

        scatters.append(pltpu.async_copy(
            gb, out_hbm.at[pl.ds(base + off, _C)], sos[g]))
    for s in scatters:
        s.wait()


@functools.partial(jax.jit, static_argnames=())
def _run(x2d, pe, table):
    mesh = plsc.VectorSubcoreMesh(core_axis_name="c", subcore_axis_name="s")
    f = pl.kernel(
        _sc_body,
        mesh=mesh,
        out_type=jax.ShapeDtypeStruct((_BATCH * _SEQ, _D), jnp.float32),
        scratch_types=[
            pltpu.VMEM((_ROWS,), jnp.int32),
            pltpu.VMEM((_C, _D), jnp.float32),
            pltpu.VMEM((_C, _D), jnp.float32),
            pltpu.VMEM((_C, _D), jnp.float32),
            pltpu.VMEM((_C, _D), jnp.float32),
            pltpu.VMEM((_ROWS, _D), jnp.float32),
            pltpu.SemaphoreType.DMA,
            pltpu.SemaphoreType.DMA,
            pltpu.SemaphoreType.DMA,
            pltpu.SemaphoreType.DMA,
            pltpu.SemaphoreType.DMA,
            pltpu.SemaphoreType.DMA,
            pltpu.SemaphoreType.DMA,
            pltpu.SemaphoreType.DMA,
            pltpu.SemaphoreType.DMA,
        ],
    )
    return f(x2d, pe, table)


def kernel(x, table):
    out = _run(x, _PE, table)
    return out.reshape(_BATCH, _SEQ, _D)


# speedup vs baseline: 1.1619x; 1.0393x over previous
import functools

import jax
import jax.numpy as jnp
import numpy as np
from jax import lax
from jax.experimental import pallas as pl
from jax.experimental.pallas import tpu as pltpu
from jax.experimental.pallas import tpu_sc as plsc

_VOCAB = 100000
_D = 128
_SEQ = 2048
_BATCH = 4
_NC = 2
_NS = 16
_NW = _NC * _NS
_ROWS = (_BATCH * _SEQ) // _NW


def _pe_table() -> np.ndarray:
    pos = np.arange(_SEQ, dtype=np.float32)[:, None]
    div = np.exp(np.arange(0, _D, 2, dtype=np.float32) * (-np.log(10000.0) / _D))
    pe = np.zeros((_SEQ, _D), dtype=np.float32)
    pe[:, 0::2] = np.sin(pos * div)
    pe[:, 1::2] = np.cos(pos * div)
    return pe


_PE = _pe_table()


_G = 4
_C = _ROWS // _G


def _sc_body(x_hbm, pe_hbm, table_hbm, out_hbm,
             idx_v, gb0, gb1, gb2, gb3, p0, p1, p2, p3, acc,
             sp, sg0, sg1, sg2, sg3, sa0, sa1, sa2, sa3,
             so0, so1, so2, so3):
    s_idx = lax.axis_index("s")
    wid = s_idx * _NC + lax.axis_index("c")
    base = wid * _ROWS
    batch = wid // (_SEQ // _ROWS)
    col = lax.rem(base, _SEQ)
    region = s_idx * _ROWS
    gbufs = (gb0, gb1, gb2, gb3)
    pbufs = (p0, p1, p2, p3)
    sgs = (sg0, sg1, sg2, sg3)
    sas = (sa0, sa1, sa2, sa3)
    sos = (so0, so1, so2, so3)
    pe_load = pltpu.async_copy(
        pe_hbm.at[pl.ds(col, _ROWS)], acc.at[pl.ds(region, _ROWS)], sp)
    pltpu.sync_copy(x_hbm.at[batch, pl.ds(col, _ROWS)], idx_v)
    gathers = [
        pltpu.async_copy(
            table_hbm.at[idx_v.at[pl.ds(g * _C, _C)]], gbufs[g], sgs[g])
        for g in range(_G)
    ]
    for g in range(_G):
        for k in range(_C // 16):
            pbufs[g][pl.ds(k * 16, 16)] = (
                region + g * _C + k * 16 + lax.iota(jnp.int32, 16))
    pe_load.wait()
    adds = []
    for g in range(_G):
        gathers[g].wait()
        adds.append(pltpu.async_copy(
            gbufs[g], acc.at[pbufs[g]], sas[g], add=True))
    outs = []
    for g in range(_G):
        adds[g].wait()
        outs.append(pltpu.async_copy(
            acc.at[pl.ds(region + g * _C, _C)],
            out_hbm.at[pl.ds(base + g * _C, _C)], sos[g]))
    for o in outs:
        o.wait()


@functools.partial(jax.jit, static_argnames=())
def _run(x2d, pe, table):
    mesh = plsc.VectorSubcoreMesh(core_axis_name="c", subcore_axis_name="s")
    f = pl.kernel(
        _sc_body,
        mesh=mesh,
        out_type=jax.ShapeDtypeStruct((_BATCH * _SEQ, _D), jnp.float32),
        scratch_types=(
            [pltpu.VMEM((_ROWS,), jnp.int32)]
            + [pltpu.VMEM((_C, _D), jnp.float32)] * _G
            + [pltpu.VMEM((_C,), jnp.int32)] * _G
            + [pltpu.VMEM_SHARED((_NS * _ROWS, _D), jnp.float32)]
            + [pltpu.SemaphoreType.DMA] * (1 + 3 * _G)
        ),
    )
    return f(x2d, pe, table)


def kernel(x, table):
    out = _run(x, _PE, table)
    return out.reshape(_BATCH, _SEQ, _D)


# speedup vs baseline: 1.1632x; 1.0011x over previous
import functools

import jax
import jax.numpy as jnp
import numpy as np
from jax import lax
from jax.experimental import pallas as pl
from jax.experimental.pallas import tpu as pltpu
from jax.experimental.pallas import tpu_sc as plsc

_VOCAB = 100000
_D = 128
_SEQ = 2048
_BATCH = 4
_NC = 2
_NS = 16
_NW = _NC * _NS
_ROWS = (_BATCH * _SEQ) // _NW


def _pe_table() -> np.ndarray:
    pos = np.arange(_SEQ, dtype=np.float32)[:, None]
    div = np.exp(np.arange(0, _D, 2, dtype=np.float32) * (-np.log(10000.0) / _D))
    pe = np.zeros((_SEQ, _D), dtype=np.float32)
    pe[:, 0::2] = np.sin(pos * div)
    pe[:, 1::2] = np.cos(pos * div)
    return pe


_PE = _pe_table()


_G = 4
_C = _ROWS // _G


def _sc_body(x_hbm, pe_hbm, table_hbm, out_hbm,
             idx_v, gb0, gb1, gb2, gb3, p0, p1, p2, p3, acc,
             sp, sg0, sg1, sg2, sg3, sa0, sa1, sa2, sa3,
             so0, so1, so2, so3):
    s_idx = lax.axis_index("s")
    wid = s_idx * _NC + lax.axis_index("c")
    base = wid * _ROWS
    batch = wid // (_SEQ // _ROWS)
    col = lax.rem(base, _SEQ)
    region = s_idx * _ROWS
    gbufs = (gb0, gb1, gb2, gb3)
    pbufs = (p0, p1, p2, p3)
    sgs = (sg0, sg1, sg2, sg3)
    sas = (sa0, sa1, sa2, sa3)
    sos = (so0, so1, so2, so3)
    pe_load = pltpu.async_copy(
        pe_hbm.at[pl.ds(col, _ROWS)], acc.at[pl.ds(region, _ROWS)], sp)
    with jax.named_scope("idx_load"):
        pltpu.sync_copy(x_hbm.at[batch, pl.ds(col, _ROWS)], idx_v)
    with jax.named_scope("gather_issue"):
        gathers = [
            pltpu.async_copy(
                table_hbm.at[idx_v.at[pl.ds(g * _C, _C)]], gbufs[g], sgs[g])
            for g in range(_G)
        ]
    with jax.named_scope("pos_setup"):
        for g in range(_G):
            for k in range(_C // 16):
                pbufs[g][pl.ds(k * 16, 16)] = (
                    region + g * _C + k * 16 + lax.iota(jnp.int32, 16))
    with jax.named_scope("pe_wait"):
        pe_load.wait()
    adds = []
    with jax.named_scope("add_phase"):
        for g in range(_G):
            gathers[g].wait()
            adds.append(pltpu.async_copy(
                gbufs[g], acc.at[pbufs[g]], sas[g], add=True))
    outs = []
    with jax.named_scope("out_phase"):
        for g in range(_G):
            adds[g].wait()
            outs.append(pltpu.async_copy(
                acc.at[pl.ds(region + g * _C, _C)],
                out_hbm.at[pl.ds(base + g * _C, _C)], sos[g]))
        for o in outs:
            o.wait()


@functools.partial(jax.jit, static_argnames=())
def _run(x2d, pe, table):
    mesh = plsc.VectorSubcoreMesh(core_axis_name="c", subcore_axis_name="s")
    f = pl.kernel(
        _sc_body,
        mesh=mesh,
        out_type=jax.ShapeDtypeStruct((_BATCH * _SEQ, _D), jnp.float32),
        scratch_types=(
            [pltpu.VMEM((_ROWS,), jnp.int32)]
            + [pltpu.VMEM((_C, _D), jnp.float32)] * _G
            + [pltpu.VMEM((_C,), jnp.int32)] * _G
            + [pltpu.VMEM_SHARED((_NS * _ROWS, _D), jnp.float32)]
            + [pltpu.SemaphoreType.DMA] * (1 + 3 * _G)
        ),
    )
    return f(x2d, pe, table)


def kernel(x, table):
    out = _run(x, _PE, table)
    return out.reshape(_BATCH, _SEQ, _D)
